# baseline (speedup 1.0000x reference)
import jax
import jax.numpy as jnp
from jax import lax
from jax.experimental import pallas as pl
from jax.experimental.pallas import tpu as pltpu
from jax.experimental.pallas import tpu_sc as plsc

_B, _S = 16384, 100
_N = _B * _S
_V, _D = 100, 256
_NC, _NS = 2, 16
_NW = _NC * _NS
_ROWS_PER_W = _N // _NW
_CHUNK = 128
_CHUNKS = _ROWS_PER_W // _CHUNK
_LANES = 16

_MAGIC = 8388608.0


def _make_sc_gather():
    mesh = plsc.VectorSubcoreMesh(core_axis_name="c", subcore_axis_name="s")

    @pl.kernel(
        out_type=jax.ShapeDtypeStruct((_N, _D), jnp.float32),
        mesh=mesh,
        scratch_types=[
            pltpu.VMEM((_V, _D), jnp.float32),
            pltpu.VMEM((_CHUNK,), jnp.float32),
            pltpu.VMEM((_CHUNK,), jnp.int32),
            pltpu.VMEM((_CHUNK, _D), jnp.float32),
            pltpu.VMEM((_CHUNK, _D), jnp.float32),
            pltpu.SemaphoreType.DMA,
            pltpu.SemaphoreType.DMA,
        ],
    )
    def sc_gather(x_hbm, pe_hbm, out_hbm, tbl_v, x_v, idx_v, rows_a, rows_b,
                  swa, swb):
        cid = lax.axis_index("c")
        sid = lax.axis_index("s")
        wid = cid * _NS + sid
        woff = wid * _ROWS_PER_W

        pltpu.sync_copy(pe_hbm, tbl_v)

        def build_chunk(c, rows):
            pltpu.sync_copy(x_hbm.at[pl.ds(woff + c * _CHUNK, _CHUNK)], x_v)
            for i in range(_CHUNK // _LANES):
                sl = pl.ds(i * _LANES, _LANES)
                y = jnp.maximum(x_v[sl], 0.01) * 100.0
                r = (y + _MAGIC) - _MAGIC
                idx_v[sl] = r.astype(jnp.int32) - 1

            @pl.loop(0, _CHUNK // _LANES)
            def _(g):
                iv = idx_v[pl.ds(g * _LANES, _LANES)]
                base = g * _LANES
                for lane in range(_LANES):
                    src = tbl_v.at[iv[lane]]
                    dst = rows.at[base + lane]
                    for j in range(_D // _LANES):
                        sl = pl.ds(j * _LANES, _LANES)
                        dst[sl] = src[sl]

        def out_slice(c):
            return out_hbm.at[pl.ds(woff + c * _CHUNK, _CHUNK)]

        def wait_write(c, rows, sw):
            pltpu.make_async_copy(rows, out_slice(c), sw).wait()

        build_chunk(0, rows_a)
        pltpu.async_copy(rows_a, out_slice(0), swa)
        build_chunk(1, rows_b)
        pltpu.async_copy(rows_b, out_slice(1), swb)

        @pl.loop(2, _CHUNKS, step=2)
        def _(c):
            wait_write(c - 2, rows_a, swa)
            build_chunk(c, rows_a)
            pltpu.async_copy(rows_a, out_slice(c), swa)
            wait_write(c - 1, rows_b, swb)
            build_chunk(c + 1, rows_b)
            pltpu.async_copy(rows_b, out_slice(c + 1), swb)

        wait_write(_CHUNKS - 2, rows_a, swa)
        wait_write(_CHUNKS - 1, rows_b, swb)

    return sc_gather


_sc_gather = _make_sc_gather()


def kernel(x, pe):
    xt = x.T.reshape(_N)
    out = _sc_gather(xt, pe)
    return out.reshape(_S, _B, _D).transpose(1, 0, 2)


# speedup vs baseline: 2.5326x; 2.5326x over previous
import jax
import jax.numpy as jnp
from jax import lax
from jax.experimental import pallas as pl
from jax.experimental.pallas import tpu as pltpu
from jax.experimental.pallas import tpu_sc as plsc

_B, _S = 16384, 100
_N = _B * _S
_V, _D = 100, 256
_NC, _NS = 2, 16
_NW = _NC * _NS
_ROWS_PER_W = _N // _NW
_CHUNK = 128
_CHUNKS = _ROWS_PER_W // _CHUNK
_LANES = 16

_MAGIC = 8388608.0


def _make_sc_gather():
    mesh = plsc.VectorSubcoreMesh(core_axis_name="c", subcore_axis_name="s")

    @pl.kernel(
        out_type=jax.ShapeDtypeStruct((_N, _D), jnp.float32),
        mesh=mesh,
        scratch_types=[
            pltpu.VMEM((_V, _D), jnp.float32),
            pltpu.VMEM((_CHUNK,), jnp.float32),
            pltpu.VMEM((_CHUNK,), jnp.int32),
            pltpu.VMEM((_CHUNK, _D), jnp.float32),
            pltpu.VMEM((_CHUNK, _D), jnp.float32),
            pltpu.SemaphoreType.DMA,
            pltpu.SemaphoreType.DMA,
        ],
    )
    def sc_gather(x_hbm, pe_hbm, out_hbm, tbl_v, x_v, idx_v, rows_a, rows_b,
                  swa, swb):
        cid = lax.axis_index("c")
        sid = lax.axis_index("s")
        wid = cid * _NS + sid
        woff = wid * _ROWS_PER_W

        pltpu.sync_copy(pe_hbm, tbl_v)

        def build_chunk(c, rows):
            pltpu.sync_copy(x_hbm.at[pl.ds(woff + c * _CHUNK, _CHUNK)], x_v)
            for i in range(_CHUNK // _LANES):
                sl = pl.ds(i * _LANES, _LANES)
                y = jnp.maximum(x_v[sl], 0.01) * 100.0
                r = (y + _MAGIC) - _MAGIC
                idx_v[sl] = r.astype(jnp.int32) - 1

            @pl.loop(0, _CHUNK // _LANES)
            def _(g):
                iv = idx_v[pl.ds(g * _LANES, _LANES)]
                base = g * _LANES
                for lane in range(_LANES):
                    src = tbl_v.at[iv[lane]]
                    dst = rows.at[base + lane]
                    vals = [src[pl.ds(j * _LANES, _LANES)]
                            for j in range(_D // _LANES)]
                    for j in range(_D // _LANES):
                        dst[pl.ds(j * _LANES, _LANES)] = vals[j]

        def out_slice(c):
            return out_hbm.at[pl.ds(woff + c * _CHUNK, _CHUNK)]

        def wait_write(c, rows, sw):
            pltpu.make_async_copy(rows, out_slice(c), sw).wait()

        build_chunk(0, rows_a)
        pltpu.async_copy(rows_a, out_slice(0), swa)
        build_chunk(1, rows_b)
        pltpu.async_copy(rows_b, out_slice(1), swb)

        @pl.loop(2, _CHUNKS, step=2)
        def _(c):
            wait_write(c - 2, rows_a, swa)
            build_chunk(c, rows_a)
            pltpu.async_copy(rows_a, out_slice(c), swa)
            wait_write(c - 1, rows_b, swb)
            build_chunk(c + 1, rows_b)
            pltpu.async_copy(rows_b, out_slice(c + 1), swb)

        wait_write(_CHUNKS - 2, rows_a, swa)
        wait_write(_CHUNKS - 1, rows_b, swb)

    return sc_gather


_sc_gather = _make_sc_gather()


def kernel(x, pe):
    xt = x.T.reshape(_N)
    out = _sc_gather(xt, pe)
    return out.reshape(_S, _B, _D).transpose(1, 0, 2)
